# baseline (speedup 1.0000x reference)
import functools

import jax
import jax.numpy as jnp
from jax import lax
from jax.experimental import pallas as pl
from jax.experimental.pallas import tpu as pltpu
from jax.experimental.pallas import tpu_sc as plsc

_EPS = 1e-5
_L = 16


def _rsqrt16(a):
    i = plsc.bitcast(a, jnp.int32)
    i = jnp.int32(0x5F3759DF) - lax.shift_right_logical(i, 1)
    y = plsc.bitcast(i, jnp.float32)
    h = a * 0.5
    for _ in range(3):
        y = y * (1.5 - h * y * y)
    return y


def _build_sc_kernel(N, E, S, n_workers, npw):
    mesh = plsc.VectorSubcoreMesh(core_axis_name="c", subcore_axis_name="s")

    @functools.partial(
        pl.kernel,
        out_type=jax.ShapeDtypeStruct((N, E), jnp.float32),
        mesh=mesh,
        scratch_types=[
            pltpu.VMEM((npw,), jnp.int32),
            pltpu.VMEM((npw, E), jnp.float32),
            pltpu.VMEM((npw, E), jnp.float32),
            pltpu.VMEM((E,), jnp.float32),
            pltpu.VMEM((E,), jnp.float32),
            pltpu.SemaphoreType.DMA,
        ],
        compiler_params=pltpu.CompilerParams(
            needs_layout_passes=False, use_tc_tiling_on_sc=False),
    )
    def emb_kernel(idx_hbm, tok_hbm, pos_hbm, gamma_hbm, beta_hbm, out_hbm,
                   idx_v, rows_v, pos_v, g_v, b_v, sem):
        wid = lax.axis_index("s") * 2 + lax.axis_index("c")
        base = wid * npw
        pos_base = lax.rem(base, S)

        pltpu.sync_copy(idx_hbm.at[pl.ds(base, npw)], idx_v)
        pltpu.sync_copy(gamma_hbm, g_v)
        pltpu.sync_copy(beta_hbm, b_v)
        pltpu.sync_copy(pos_hbm.at[pl.ds(pos_base, npw)], pos_v)
        pltpu.async_copy(tok_hbm.at[idx_v], rows_v, sem).wait()

        nk = E // _L
        g = [g_v[pl.ds(k * _L, _L)] for k in range(nk)]
        b = [b_v[pl.ds(k * _L, _L)] for k in range(nk)]

        def body(t, carry):
            x = [rows_v[t, pl.ds(k * _L, _L)] + pos_v[t, pl.ds(k * _L, _L)]
                 for k in range(nk)]
            s = x[0]
            sq = x[0] * x[0]
            for k in range(1, nk):
                s = s + x[k]
                sq = sq + x[k] * x[k]
            tot = jnp.sum(s)
            tot2 = jnp.sum(sq)
            mean = tot * (1.0 / E)
            var = tot2 * (1.0 / E) - mean * mean
            meanv = jnp.full((_L,), mean, jnp.float32)
            rstd = _rsqrt16(jnp.full((_L,), var + _EPS, jnp.float32))
            for k in range(nk):
                rows_v[t, pl.ds(k * _L, _L)] = (x[k] - meanv) * rstd * g[k] + b[k]
            return carry

        lax.fori_loop(0, npw, body, 0)
        pltpu.sync_copy(rows_v, out_hbm.at[pl.ds(base, npw)])

    return emb_kernel


def kernel(input_tensor, token_table, pos_table, gamma, beta):
    B, S = input_tensor.shape
    V, E = token_table.shape
    N = B * S
    n_workers = 32
    npw = N // n_workers
    idx = input_tensor.reshape(N).astype(jnp.int32)
    f = _build_sc_kernel(N, E, S, n_workers, npw)
    out = f(idx, token_table, pos_table, gamma, beta)
    return out.reshape(B, S, E)


# speedup vs baseline: 4.2560x; 4.2560x over previous
import functools

import jax
import jax.numpy as jnp
from jax import lax
from jax.experimental import pallas as pl
from jax.experimental.pallas import tpu as pltpu
from jax.experimental.pallas import tpu_sc as plsc

_EPS = 1e-5
_L = 16
_GRP = 4


def _rsqrt16(a):
    i = plsc.bitcast(a, jnp.int32)
    i = jnp.int32(0x5F3759DF) - lax.shift_right_logical(i, 1)
    y = plsc.bitcast(i, jnp.float32)
    h = a * 0.5
    for _ in range(3):
        y = y * (1.5 - h * y * y)
    return y


def _build_sc_kernel(B, S, V, E, npw):
    mesh = plsc.VectorSubcoreMesh(core_axis_name="c", subcore_axis_name="s")
    N = B * S
    nk = E // _L
    tpi = 2 * _GRP
    cmax = ((V - 1) // 128) * 128

    @functools.partial(
        pl.kernel,
        out_type=jax.ShapeDtypeStruct((N, 128), jnp.float32),
        mesh=mesh,
        scratch_types=[
            pltpu.VMEM((npw + 2 * tpi,), jnp.int32),
            [pltpu.VMEM((E, 128), jnp.float32) for _ in range(2 * _GRP)],
            pltpu.VMEM((npw, 128), jnp.float32),
            pltpu.VMEM((E, 128), jnp.float32),
            pltpu.VMEM((E, 128), jnp.float32),
            pltpu.VMEM((E,), jnp.float32),
            pltpu.VMEM((E,), jnp.float32),
            pltpu.SemaphoreType.DMA,
            pltpu.SemaphoreType.DMA,
            pltpu.SemaphoreType.DMA,
        ],
        compiler_params=pltpu.CompilerParams(
            needs_layout_passes=False, use_tc_tiling_on_sc=True,
            disable_bounds_checks=True),
    )
    def emb_kernel(idx_hbm, tabT_hbm, posT_hbm, gamma_hbm, beta_hbm, out_hbm,
                   idx_v, bufs, outb, posA, posB, g_v, b_v,
                   semA, semB, semp):
        wid = lax.axis_index("s") * 2 + lax.axis_index("c")
        base = pl.multiple_of(wid * npw, npw)
        p0 = pl.multiple_of(lax.rem(base, S), 128)

        pltpu.sync_copy(idx_hbm.at[pl.ds(base, npw)], idx_v.at[pl.ds(0, npw)])
        cp1 = pltpu.async_copy(posT_hbm.at[:, pl.ds(p0, 128)], posA, semp)
        cp2 = pltpu.async_copy(posT_hbm.at[:, pl.ds(p0 + 128, 128)], posB, semp)
        pltpu.sync_copy(gamma_hbm, g_v)
        pltpu.sync_copy(beta_hbm, b_v)
        cp1.wait()
        cp2.wait()

        rows = [lax.iota(jnp.int32, _L) + k * _L for k in range(nk)]
        g = [g_v[pl.ds(k * _L, _L)] for k in range(nk)]
        b = [b_v[pl.ds(k * _L, _L)] for k in range(nk)]

        def fire(s, slot, sem):
            cc = pl.multiple_of(
                lax.max(0, lax.min(lax.div(s, 128) * 128, cmax)), 128)
            return pltpu.async_copy(
                tabT_hbm.at[:, pl.ds(cc, 128)], bufs[slot], sem)

        def drain(slot, sem):
            pltpu.make_async_copy(
                tabT_hbm.at[:, pl.ds(0, 128)], bufs[slot], sem).wait()

        def do_token(buf, t, s, pos_buf):
            j = jnp.full((_L,), lax.rem(s, 128), jnp.int32)
            pcol = jnp.full((_L,), lax.rem(t, 128), jnp.int32)
            x = []
            for q in range(nk):
                tok = plsc.load_gather(buf, [rows[q], j])
                pv = plsc.load_gather(pos_buf, [rows[q], pcol])
                x.append(tok + pv)
            tot = x[0] + x[1] + x[2] + x[3]
            sq = x[0] * x[0] + x[1] * x[1] + x[2] * x[2] + x[3] * x[3]
            sm = jnp.sum(tot)
            sq = jnp.sum(sq)
            mean = sm * (1.0 / E)
            var = sq * (1.0 / E) - mean * mean
            meanv = jnp.full((_L,), mean, jnp.float32)
            rstd = _rsqrt16(jnp.full((_L,), var + _EPS, jnp.float32))
            for q in range(nk):
                outb[t, pl.ds(q * _L, _L)] = (x[q] - meanv) * rstd * g[q] + b[q]

        sv0 = idx_v[pl.ds(0, _L)]
        for k in range(_GRP):
            fire(sv0[k], k, semA)
        for k in range(_GRP):
            fire(sv0[_GRP + k], _GRP + k, semB)

        def make_chunk(pos_buf):
            def chunk(c, carry):
                t0 = c * tpi
                sv = idx_v[pl.ds(t0, _L)]
                svn = idx_v[pl.ds(t0 + tpi, _L)]
                for k in range(_GRP):
                    drain(k, semA)
                for k in range(_GRP):
                    do_token(bufs[k], t0 + k, sv[k], pos_buf)
                for k in range(_GRP):
                    fire(svn[k], k, semA)
                for k in range(_GRP):
                    drain(_GRP + k, semB)
                for k in range(_GRP):
                    do_token(bufs[_GRP + k], t0 + _GRP + k, sv[_GRP + k],
                             pos_buf)
                for k in range(_GRP):
                    fire(svn[_GRP + k], _GRP + k, semB)
                return carry
            return chunk

        nhalf = (npw // 2) // tpi
        lax.fori_loop(0, nhalf, make_chunk(posA), 0)
        lax.fori_loop(nhalf, 2 * nhalf, make_chunk(posB), 0)

        for k in range(_GRP):
            drain(k, semA)
        for k in range(_GRP):
            drain(_GRP + k, semB)

        pltpu.sync_copy(outb, out_hbm.at[pl.ds(base, npw), :])

    return emb_kernel


def kernel(input_tensor, token_table, pos_table, gamma, beta):
    B, S = input_tensor.shape
    V, E = token_table.shape
    N = B * S
    npw = N // 32
    idx = input_tensor.reshape(N).astype(jnp.int32)
    f = _build_sc_kernel(B, S, V, E, npw)
    out = f(idx, token_table.T, pos_table.T, gamma, beta)
    return out[:, :E].reshape(B, S, E)


# speedup vs baseline: 4.3324x; 1.0180x over previous
import functools

import jax
import jax.numpy as jnp
from jax import lax
from jax.experimental import pallas as pl
from jax.experimental.pallas import tpu as pltpu
from jax.experimental.pallas import tpu_sc as plsc

_EPS = 1e-5
_L = 16
_GRP = 4


def _rsqrt16(a):
    i = plsc.bitcast(a, jnp.int32)
    i = jnp.int32(0x5F3759DF) - lax.shift_right_logical(i, 1)
    y = plsc.bitcast(i, jnp.float32)
    h = a * 0.5
    for _ in range(3):
        y = y * (1.5 - h * y * y)
    return y


def _build_sc_kernel(B, S, V, E, npw):
    mesh = plsc.VectorSubcoreMesh(core_axis_name="c", subcore_axis_name="s")
    N = B * S
    nk = E // _L
    tpi = 2 * _GRP
    cmax = ((V - 1) // 128) * 128

    @functools.partial(
        pl.kernel,
        out_type=jax.ShapeDtypeStruct((B, E, S), jnp.float32),
        mesh=mesh,
        scratch_types=[
            pltpu.VMEM((npw + 2 * tpi,), jnp.int32),
            [pltpu.VMEM((E, 128), jnp.float32) for _ in range(2 * _GRP)],
            pltpu.VMEM((E, 128), jnp.float32),
            pltpu.VMEM((E, 128), jnp.float32),
            pltpu.VMEM((E, 128), jnp.float32),
            pltpu.VMEM((E, 128), jnp.float32),
            pltpu.VMEM((E,), jnp.float32),
            pltpu.VMEM((E,), jnp.float32),
            pltpu.SemaphoreType.DMA,
            pltpu.SemaphoreType.DMA,
            pltpu.SemaphoreType.DMA,
        ],
        compiler_params=pltpu.CompilerParams(
            needs_layout_passes=False, use_tc_tiling_on_sc=True,
            disable_bounds_checks=True),
    )
    def emb_kernel(idx_hbm, tabT_hbm, posT_hbm, gamma_hbm, beta_hbm, out_hbm,
                   idx_v, bufs, outA, outB, posA, posB, g_v, b_v,
                   semA, semB, semp):
        wid = lax.axis_index("s") * 2 + lax.axis_index("c")
        base = pl.multiple_of(wid * npw, npw)
        b0 = lax.div(base, S)
        p0 = pl.multiple_of(lax.rem(base, S), 128)

        pltpu.sync_copy(idx_hbm.at[pl.ds(base, npw)], idx_v.at[pl.ds(0, npw)])
        cp1 = pltpu.async_copy(posT_hbm.at[:, pl.ds(p0, 128)], posA, semp)
        cp2 = pltpu.async_copy(posT_hbm.at[:, pl.ds(p0 + 128, 128)], posB, semp)
        pltpu.sync_copy(gamma_hbm, g_v)
        pltpu.sync_copy(beta_hbm, b_v)
        cp1.wait()
        cp2.wait()

        rows = [lax.iota(jnp.int32, _L) + k * _L for k in range(nk)]
        g = [g_v[pl.ds(k * _L, _L)] for k in range(nk)]
        b = [b_v[pl.ds(k * _L, _L)] for k in range(nk)]

        def fire(s, slot, sem):
            cc = pl.multiple_of(
                lax.max(0, lax.min(lax.div(s, 128) * 128, cmax)), 128)
            return pltpu.async_copy(
                tabT_hbm.at[:, pl.ds(cc, 128)], bufs[slot], sem)

        def drain(slot, sem):
            pltpu.make_async_copy(
                tabT_hbm.at[:, pl.ds(0, 128)], bufs[slot], sem).wait()

        def do_token(buf, t, s, pos_buf, out_buf):
            j = jnp.full((_L,), lax.rem(s, 128), jnp.int32)
            pcol = jnp.full((_L,), lax.rem(t, 128), jnp.int32)
            x = []
            for q in range(nk):
                tok = plsc.load_gather(buf, [rows[q], j])
                pv = plsc.load_gather(pos_buf, [rows[q], pcol])
                x.append(tok + pv)
            tot = x[0] + x[1] + x[2] + x[3]
            sq = x[0] * x[0] + x[1] * x[1] + x[2] * x[2] + x[3] * x[3]
            sm = jnp.sum(tot)
            sq = jnp.sum(sq)
            mean = sm * (1.0 / E)
            var = sq * (1.0 / E) - mean * mean
            meanv = jnp.full((_L,), mean, jnp.float32)
            rstd = _rsqrt16(jnp.full((_L,), var + _EPS, jnp.float32))
            for q in range(nk):
                val = (x[q] - meanv) * rstd * g[q] + b[q]
                plsc.store_scatter(out_buf, [rows[q], pcol], val)

        sv0 = idx_v[pl.ds(0, _L)]
        for k in range(_GRP):
            fire(sv0[k], k, semA)
        for k in range(_GRP):
            fire(sv0[_GRP + k], _GRP + k, semB)

        def make_chunk(pos_buf, out_buf):
            def chunk(c, carry):
                t0 = c * tpi
                sv = idx_v[pl.ds(t0, _L)]
                svn = idx_v[pl.ds(t0 + tpi, _L)]
                for k in range(_GRP):
                    drain(k, semA)
                for k in range(_GRP):
                    do_token(bufs[k], t0 + k, sv[k], pos_buf, out_buf)
                for k in range(_GRP):
                    fire(svn[k], k, semA)
                for k in range(_GRP):
                    drain(_GRP + k, semB)
                for k in range(_GRP):
                    do_token(bufs[_GRP + k], t0 + _GRP + k, sv[_GRP + k],
                             pos_buf, out_buf)
                for k in range(_GRP):
                    fire(svn[_GRP + k], _GRP + k, semB)
                return carry
            return chunk

        nhalf = (npw // 2) // tpi
        lax.fori_loop(0, nhalf, make_chunk(posA, outA), 0)
        out_cp = pltpu.async_copy(
            outA, out_hbm.at[b0, :, pl.ds(p0, 128)], semp)
        lax.fori_loop(nhalf, 2 * nhalf, make_chunk(posB, outB), 0)

        for k in range(_GRP):
            drain(k, semA)
        for k in range(_GRP):
            drain(_GRP + k, semB)

        out_cp.wait()
        pltpu.sync_copy(outB, out_hbm.at[b0, :, pl.ds(p0 + 128, 128)])

    return emb_kernel


def kernel(input_tensor, token_table, pos_table, gamma, beta):
    B, S = input_tensor.shape
    V, E = token_table.shape
    N = B * S
    npw = N // 32
    idx = input_tensor.reshape(N).astype(jnp.int32)
    f = _build_sc_kernel(B, S, V, E, npw)
    out = f(idx, token_table.T, pos_table.T, gamma, beta)
    return jnp.transpose(out, (0, 2, 1))
